# Initial kernel scaffold; baseline (speedup 1.0000x reference)
import jax
import jax.numpy as jnp
from jax.experimental import pallas as pl


def kernel(tokens, timestamps, channel_embed, pos_embed, month_table):
    raise NotImplementedError("write your pallas kernel here")


# speedup vs baseline: 2.5569x; 2.5569x over previous
import jax
import jax.numpy as jnp
from jax.experimental import pallas as pl
from jax.experimental.pallas import tpu as pltpu


def _embed_add_kernel(ts_ref, tokens_ref, ch_ref, pos_ref, mt_ref, out_ref):
    ib = pl.program_id(0)
    it = pl.program_id(1)
    month = ts_ref[ib, it, 1]
    add = ch_ref[...] + pos_ref[it, :][None, :] + mt_ref[month, :][None, :]
    out_ref[...] = tokens_ref[...] + add[None, None, None, None, :, :]


def kernel(tokens, timestamps, channel_embed, pos_embed, month_table):
    b, h, w, t, bs, d = tokens.shape
    n = d // 4
    ch_p = jnp.pad(channel_embed, ((0, 0), (0, d - n)))
    pos_p = jnp.pad(pos_embed[:t], ((0, 0), (n, d - 2 * n)))
    mt_p = jnp.pad(month_table, ((0, 0), (2 * n, d - 3 * n)))

    grid = (b, t)
    return pl.pallas_call(
        _embed_add_kernel,
        grid=grid,
        in_specs=[
            pl.BlockSpec(memory_space=pltpu.SMEM),
            pl.BlockSpec((1, h, w, 1, bs, d), lambda ib, it: (ib, 0, 0, it, 0, 0)),
            pl.BlockSpec((bs, d), lambda ib, it: (0, 0)),
            pl.BlockSpec((t, d), lambda ib, it: (0, 0)),
            pl.BlockSpec((12, d), lambda ib, it: (0, 0)),
        ],
        out_specs=pl.BlockSpec((1, h, w, 1, bs, d), lambda ib, it: (ib, 0, 0, it, 0, 0)),
        out_shape=jax.ShapeDtypeStruct(tokens.shape, tokens.dtype),
        compiler_params=pltpu.CompilerParams(
            dimension_semantics=("arbitrary", "arbitrary"),
        ),
    )(timestamps, tokens, ch_p, pos_p, mt_p)
